# Initial kernel scaffold; baseline (speedup 1.0000x reference)
import jax
import jax.numpy as jnp
from jax.experimental import pallas as pl


def kernel(tokens, table):
    raise NotImplementedError("write your pallas kernel here")


# speedup vs baseline: 1.3067x; 1.3067x over previous
import functools

import jax
import jax.numpy as jnp
from jax import lax
from jax.experimental import pallas as pl
from jax.experimental.pallas import tpu as pltpu
from jax.experimental.pallas import tpu_sc as plsc

_NC = 2
_NS = 16
_NW = _NC * _NS
_L = 16


def _make_kernel(S, B, V, E):
    assert B % _NW == 0
    nb = B // _NW
    assert E % _L == 0
    ne = E // _L
    chunks = []
    off = 0
    while off < S:
        n = min(128, S - off)
        assert n % 8 == 0 and off % 8 == 0
        chunks.append((off, n))
        off += n

    mesh = plsc.VectorSubcoreMesh(core_axis_name="c", subcore_axis_name="s")

    @functools.partial(
        pl.kernel,
        out_type=jax.ShapeDtypeStruct((B, E, S), jnp.float32),
        mesh=mesh,
        scratch_types=[
            pltpu.VMEM((S,), jnp.int32),
            pltpu.VMEM((S, E), jnp.float32),
            pltpu.VMEM((E, S), jnp.float32),
            pltpu.SemaphoreType.DMA,
        ],
        compiler_params=pltpu.CompilerParams(
            use_tc_tiling_on_sc=False, needs_layout_passes=False
        ),
    )
    def k(tok_hbm, table_hbm, out_hbm, idx_v, rows_v, tv, sem):
        wid = lax.axis_index("s") * _NC + lax.axis_index("c")
        iota = lax.iota(jnp.int32, _L)

        def per_batch(bi, carry):
            b = bi * _NW + wid
            pltpu.sync_copy(tok_hbm.at[b], idx_v)
            copies = [
                pltpu.make_async_copy(
                    table_hbm.at[idx_v.at[pl.ds(off, n)]],
                    rows_v.at[pl.ds(off, n)],
                    sem,
                )
                for off, n in chunks
            ]
            for cp in copies:
                cp.start()
            for cp in copies:
                cp.wait()

            def per_s(si, c2):
                svec = jnp.full((_L,), si, jnp.int32)
                for e0 in range(ne):
                    vals = rows_v[si, pl.ds(e0 * _L, _L)]
                    plsc.store_scatter(tv, [e0 * _L + iota, svec], vals)
                return c2

            lax.fori_loop(0, S, per_s, 0, unroll=False)
            pltpu.sync_copy(tv, out_hbm.at[b])
            return carry

        lax.fori_loop(0, nb, per_batch, 0, unroll=False)

    return k


def kernel(tokens, table):
    S, B = tokens.shape
    V, E = table.shape
    tok_t = jnp.transpose(tokens).astype(jnp.int32)
    k = _make_kernel(S, B, V, E)
    return k(tok_t, table)
